# Initial kernel scaffold; baseline (speedup 1.0000x reference)
import jax
import jax.numpy as jnp
from jax.experimental import pallas as pl


def kernel(content_tokens, user_type_idx, user_handle_idx, content_table, user_table, W1, b1, W2, b2, ln_gamma, ln_beta, W_out):
    raise NotImplementedError("write your pallas kernel here")


# speedup vs baseline: 2.4144x; 2.4144x over previous
import functools

import jax
import jax.numpy as jnp
from jax.experimental import pallas as pl
from jax.experimental.pallas import tpu as pltpu
from jax.experimental.pallas import tpu_sc as plsc

B = 4096
L = 50
D_EMB = 64
N_TYPES = 9
LN_EPS = 1e-3

_NC = 2
_NS = 16
_NW = _NC * _NS
_B_BLK = 256
_G = 128


def _sc_gather_all(content_tab2, tok_idx, user_tab2, handle_idx,
                   type_table, type_idx):
    n_tok = tok_idx.size
    tok_pw = n_tok // _NW
    n_ch = tok_pw // _G
    u_pw = B // _NW
    d2 = content_tab2.shape[1]
    d_t = type_table.shape[1]
    tok3 = tok_idx.reshape(_NW, n_ch, _G)
    hand2 = handle_idx.reshape(_NW, u_pw)
    typ2 = type_idx.reshape(_NW, u_pw)
    mesh = plsc.VectorSubcoreMesh(core_axis_name="c", subcore_axis_name="s")

    @functools.partial(
        pl.kernel, mesh=mesh,
        out_type=(jax.ShapeDtypeStruct((n_tok, d2), jnp.float32),
                  jax.ShapeDtypeStruct((B, d2), jnp.float32),
                  jax.ShapeDtypeStruct((B, d_t), jnp.float32)),
        scratch_types=[pltpu.VMEM((n_ch, _G), jnp.int32),
                       pltpu.VMEM((_G, d2), jnp.float32),
                       pltpu.VMEM((u_pw,), jnp.int32),
                       pltpu.VMEM((u_pw, d2), jnp.float32),
                       pltpu.VMEM((u_pw, d_t), jnp.float32),
                       pltpu.SemaphoreType.DMA])
    def gather_kernel(ctab, cidx, utab, uidx, ttab, tidx, cout, uout, tout,
                      cidx_v, crows_v, sidx_v, urows_v, trows_v, sem):
        wid = jax.lax.axis_index("s") * _NC + jax.lax.axis_index("c")
        cbase = wid * tok_pw
        pltpu.sync_copy(cidx.at[wid], cidx_v)

        @pl.loop(0, n_ch)
        def _(c):
            pltpu.async_copy(ctab.at[cidx_v.at[c]], crows_v, sem).wait()
            pltpu.sync_copy(crows_v, cout.at[pl.ds(cbase + c * _G, _G)])

        ubase = wid * u_pw
        pltpu.sync_copy(uidx.at[wid], sidx_v)
        pltpu.async_copy(utab.at[sidx_v], urows_v, sem).wait()
        pltpu.sync_copy(urows_v, uout.at[pl.ds(ubase, u_pw)])

        pltpu.sync_copy(tidx.at[wid], sidx_v)
        pltpu.async_copy(ttab.at[sidx_v], trows_v, sem).wait()
        pltpu.sync_copy(trows_v, tout.at[pl.ds(ubase, u_pw)])

    return gather_kernel(content_tab2, tok3, user_tab2, hand2,
                         type_table, typ2)


def _tc_mlp_kernel(cont_ref, vis_ref, typeb_ref, w1a_ref, w1c_ref, w2_ref,
                   b2_ref, g_ref, beta_ref, wout_ref, out_ref):
    x = cont_ref[...]
    qm = jnp.mean(x, axis=1)[:, :D_EMB]
    h1 = jnp.dot(qm, w1a_ref[...], preferred_element_type=jnp.float32)
    h1 += jnp.dot(vis_ref[...][:, :D_EMB], w1c_ref[...],
                  preferred_element_type=jnp.float32)
    h1 += typeb_ref[...]
    h1 = jnp.maximum(h1, 0.0)
    h2 = jnp.dot(h1, w2_ref[...], preferred_element_type=jnp.float32)
    h2 += b2_ref[...]
    h2 = jnp.maximum(h2, 0.0)
    mu = jnp.mean(h2, axis=-1, keepdims=True)
    dev = h2 - mu
    var = jnp.mean(dev * dev, axis=-1, keepdims=True)
    hn = dev * jax.lax.rsqrt(var + LN_EPS) * g_ref[...] + beta_ref[...]
    out_ref[...] = jnp.dot(hn, wout_ref[...], preferred_element_type=jnp.float32)


def kernel(content_tokens, user_type_idx, user_handle_idx, content_table,
           user_table, W1, b1, W2, b2, ln_gamma, ln_beta, W_out):
    n_out = W_out.shape[1]
    type_table = W1[D_EMB:D_EMB + N_TYPES] + b1[None, :]
    ctab2 = jnp.concatenate([content_table, content_table], axis=1)
    utab2 = jnp.concatenate([user_table, user_table], axis=1)
    cont, vis, typeb = _sc_gather_all(
        ctab2, content_tokens.reshape(B * L), utab2,
        user_handle_idx, type_table, user_type_idx)

    cont3 = cont.reshape(B, L, 2 * D_EMB)
    w1a = W1[:D_EMB]
    w1c = W1[D_EMB + N_TYPES:]

    grid = (B // _B_BLK,)
    return pl.pallas_call(
        _tc_mlp_kernel,
        grid=grid,
        in_specs=[
            pl.BlockSpec((_B_BLK, L, 2 * D_EMB), lambda i: (i, 0, 0)),
            pl.BlockSpec((_B_BLK, 2 * D_EMB), lambda i: (i, 0)),
            pl.BlockSpec((_B_BLK, W1.shape[1]), lambda i: (i, 0)),
            pl.BlockSpec(w1a.shape, lambda i: (0, 0)),
            pl.BlockSpec(w1c.shape, lambda i: (0, 0)),
            pl.BlockSpec(W2.shape, lambda i: (0, 0)),
            pl.BlockSpec((1, W2.shape[1]), lambda i: (0, 0)),
            pl.BlockSpec((1, W2.shape[1]), lambda i: (0, 0)),
            pl.BlockSpec((1, W2.shape[1]), lambda i: (0, 0)),
            pl.BlockSpec(W_out.shape, lambda i: (0, 0)),
        ],
        out_specs=pl.BlockSpec((_B_BLK, n_out), lambda i: (i, 0)),
        out_shape=jax.ShapeDtypeStruct((B, n_out), jnp.float32),
    )(cont3, vis, typeb, w1a, w1c, W2, b2.reshape(1, -1),
      ln_gamma.reshape(1, -1), ln_beta.reshape(1, -1), W_out)


# speedup vs baseline: 3.1812x; 1.3176x over previous
import functools

import jax
import jax.numpy as jnp
from jax.experimental import pallas as pl
from jax.experimental.pallas import tpu as pltpu
from jax.experimental.pallas import tpu_sc as plsc

B = 4096
L = 50
D_EMB = 64
N_TYPES = 9
LN_EPS = 1e-3

_NC = 2
_NS = 16
_NW = _NC * _NS
_B_BLK = 256
_G = 128


def _sc_gather_all(content_tab2, tok_idx, user_tab2, handle_idx,
                   type_table, type_idx):
    n_tok = tok_idx.size
    tok_pw = n_tok // _NW
    n_ch = tok_pw // _G
    u_pw = B // _NW
    d2 = content_tab2.shape[1]
    d_t = type_table.shape[1]
    tok3 = tok_idx.reshape(_NW, n_ch, _G)
    hand2 = handle_idx.reshape(_NW, u_pw)
    typ2 = type_idx.reshape(_NW, u_pw)
    mesh = plsc.VectorSubcoreMesh(core_axis_name="c", subcore_axis_name="s")

    @functools.partial(
        pl.kernel, mesh=mesh,
        out_type=(jax.ShapeDtypeStruct((n_tok, d2), jnp.float32),
                  jax.ShapeDtypeStruct((B, d2), jnp.float32),
                  jax.ShapeDtypeStruct((B, d_t), jnp.float32)),
        scratch_types=[pltpu.VMEM((n_ch, _G), jnp.int32),
                       pltpu.VMEM((_G, d2), jnp.float32),
                       pltpu.VMEM((u_pw,), jnp.int32),
                       pltpu.VMEM((u_pw, d2), jnp.float32),
                       pltpu.VMEM((u_pw, d_t), jnp.float32),
                       pltpu.SemaphoreType.DMA])
    def gather_kernel(ctab, cidx, utab, uidx, ttab, tidx, cout, uout, tout,
                      cidx_v, crows_v, sidx_v, urows_v, trows_v, sem):
        wid = jax.lax.axis_index("s") * _NC + jax.lax.axis_index("c")
        cbase = wid * tok_pw
        pltpu.sync_copy(cidx.at[wid], cidx_v)

        @pl.loop(0, n_ch)
        def _(c):
            pltpu.async_copy(ctab.at[cidx_v.at[c]], crows_v, sem).wait()
            pltpu.sync_copy(crows_v, cout.at[pl.ds(cbase + c * _G, _G)])

        ubase = wid * u_pw
        pltpu.sync_copy(uidx.at[wid], sidx_v)
        pltpu.async_copy(utab.at[sidx_v], urows_v, sem).wait()
        pltpu.sync_copy(urows_v, uout.at[pl.ds(ubase, u_pw)])

        pltpu.sync_copy(tidx.at[wid], sidx_v)
        pltpu.async_copy(ttab.at[sidx_v], trows_v, sem).wait()
        pltpu.sync_copy(trows_v, tout.at[pl.ds(ubase, u_pw)])

    return gather_kernel(content_tab2, tok3, user_tab2, hand2,
                         type_table, typ2)


def _tc_mlp_kernel(cont_ref, vis_ref, typeb_ref, w1a_ref, w1c_ref, w2_ref,
                   b2_ref, g_ref, beta_ref, wout_ref, out_ref):
    x = cont_ref[...]
    qm = jnp.mean(x, axis=0)[:, :D_EMB]
    h1 = jnp.dot(qm, w1a_ref[...], preferred_element_type=jnp.float32)
    h1 += jnp.dot(vis_ref[...][:, :D_EMB], w1c_ref[...],
                  preferred_element_type=jnp.float32)
    h1 += typeb_ref[...]
    h1 = jnp.maximum(h1, 0.0)
    h2 = jnp.dot(h1, w2_ref[...], preferred_element_type=jnp.float32)
    h2 += b2_ref[...]
    h2 = jnp.maximum(h2, 0.0)
    mu = jnp.mean(h2, axis=-1, keepdims=True)
    dev = h2 - mu
    var = jnp.mean(dev * dev, axis=-1, keepdims=True)
    hn = dev * jax.lax.rsqrt(var + LN_EPS) * g_ref[...] + beta_ref[...]
    out_ref[...] = jnp.dot(hn, wout_ref[...], preferred_element_type=jnp.float32)


def kernel(content_tokens, user_type_idx, user_handle_idx, content_table,
           user_table, W1, b1, W2, b2, ln_gamma, ln_beta, W_out):
    n_out = W_out.shape[1]
    type_table = W1[D_EMB:D_EMB + N_TYPES] + b1[None, :]
    ctab2 = jnp.concatenate([content_table, content_table], axis=1)
    utab2 = jnp.concatenate([user_table, user_table], axis=1)
    cont, vis, typeb = _sc_gather_all(
        ctab2, content_tokens.T.reshape(B * L), utab2,
        user_handle_idx, type_table, user_type_idx)

    cont3 = cont.reshape(L, B, 2 * D_EMB)
    w1a = W1[:D_EMB]
    w1c = W1[D_EMB + N_TYPES:]

    grid = (B // _B_BLK,)
    return pl.pallas_call(
        _tc_mlp_kernel,
        grid=grid,
        in_specs=[
            pl.BlockSpec((L, _B_BLK, 2 * D_EMB), lambda i: (0, i, 0)),
            pl.BlockSpec((_B_BLK, 2 * D_EMB), lambda i: (i, 0)),
            pl.BlockSpec((_B_BLK, W1.shape[1]), lambda i: (i, 0)),
            pl.BlockSpec(w1a.shape, lambda i: (0, 0)),
            pl.BlockSpec(w1c.shape, lambda i: (0, 0)),
            pl.BlockSpec(W2.shape, lambda i: (0, 0)),
            pl.BlockSpec((1, W2.shape[1]), lambda i: (0, 0)),
            pl.BlockSpec((1, W2.shape[1]), lambda i: (0, 0)),
            pl.BlockSpec((1, W2.shape[1]), lambda i: (0, 0)),
            pl.BlockSpec(W_out.shape, lambda i: (0, 0)),
        ],
        out_specs=pl.BlockSpec((_B_BLK, n_out), lambda i: (i, 0)),
        out_shape=jax.ShapeDtypeStruct((B, n_out), jnp.float32),
    )(cont3, vis, typeb, w1a, w1c, W2, b2.reshape(1, -1),
      ln_gamma.reshape(1, -1), ln_beta.reshape(1, -1), W_out)
